# baseline (speedup 1.0000x reference)
import jax
import jax.numpy as jnp
import numpy as np
from jax.experimental import pallas as pl

B, C, H, W_SPATIAL = 1024, 768, 14, 14
HW = H * W_SPATIAL
K_TOTAL = C * HW
LANES = 128
SUB = K_TOTAL // LANES
NUM_CLUSTER = 10
BLOCK_B = 64
BLOCK_S = SUB // 3
GRID_I = B // BLOCK_B
GRID_S = SUB // BLOCK_S

_s = np.arange(SUB)
_BASE = (LANES * _s) // HW
_NEXT = np.minimum(_BASE + 1, C - 1)
_flat = LANES * _s[:, None] + np.arange(LANES)[None, :]
_MASK0 = (_flat // HW == _BASE[:, None]).astype(np.float32)


def _sem_head_kernel(x_ref, m0_ref, t0_ref, t1_ref, b_ref, out_ref):
    s = pl.program_id(1)

    @pl.when(s == 0)
    def _init():
        out_ref[...] = jnp.broadcast_to(b_ref[...], (BLOCK_B, NUM_CLUSTER))

    x = x_ref[...]
    m0 = m0_ref[...]
    p0 = jnp.sum(x * m0[None, :, :], axis=2)
    ps = jnp.sum(x, axis=2)
    p1 = ps - p0
    acc = jax.lax.dot_general(
        p0, t0_ref[...],
        dimension_numbers=(((1,), (0,)), ((), ())),
        preferred_element_type=jnp.float32,
    ) + jax.lax.dot_general(
        p1, t1_ref[...],
        dimension_numbers=(((1,), (0,)), ((), ())),
        preferred_element_type=jnp.float32,
    )
    out_ref[...] += acc


@jax.jit
def kernel(fea, W, b):
    x3 = fea.reshape(B, SUB, LANES)
    wt = W.T * (1.0 / HW)
    t0 = wt[_BASE]
    t1 = wt[_NEXT]
    m0 = jnp.asarray(_MASK0)
    b2 = b.reshape(1, NUM_CLUSTER)
    return pl.pallas_call(
        _sem_head_kernel,
        grid=(GRID_I, GRID_S),
        in_specs=[
            pl.BlockSpec((BLOCK_B, BLOCK_S, LANES), lambda i, s: (i, s, 0)),
            pl.BlockSpec((BLOCK_S, LANES), lambda i, s: (s, 0)),
            pl.BlockSpec((BLOCK_S, NUM_CLUSTER), lambda i, s: (s, 0)),
            pl.BlockSpec((BLOCK_S, NUM_CLUSTER), lambda i, s: (s, 0)),
            pl.BlockSpec((1, NUM_CLUSTER), lambda i, s: (0, 0)),
        ],
        out_specs=pl.BlockSpec((BLOCK_B, NUM_CLUSTER), lambda i, s: (i, 0)),
        out_shape=jax.ShapeDtypeStruct((B, NUM_CLUSTER), jnp.float32),
    )(x3, m0, t0, t1, b2)


# speedup vs baseline: 10.5547x; 10.5547x over previous
import jax
import jax.numpy as jnp
from jax.experimental import pallas as pl
from jax.experimental.pallas import tpu as pltpu

B, C, H, W_SPATIAL = 1024, 768, 14, 14
HW = H * W_SPATIAL
NUM_CLUSTER = 10
BLOCK_P = 4
GRID = HW // BLOCK_P


def _sem_head_kernel(x_ref, w_ref, b_ref, out_ref, acc_ref):
    i = pl.program_id(0)
    psum = jnp.sum(x_ref[...], axis=0)

    @pl.when(i == 0)
    def _init():
        acc_ref[...] = psum

    @pl.when(i > 0)
    def _accum():
        acc_ref[...] += psum

    @pl.when(i == GRID - 1)
    def _finish():
        feat = acc_ref[...] * (1.0 / HW)
        score = jax.lax.dot_general(
            feat, w_ref[...],
            dimension_numbers=(((1,), (1,)), ((), ())),
            preferred_element_type=jnp.float32,
        )
        out_ref[...] = score + b_ref[...]


@jax.jit
def kernel(fea, W, b):
    planes = fea.transpose(2, 3, 0, 1).reshape(HW, B, C)
    b2 = b.reshape(1, NUM_CLUSTER)
    return pl.pallas_call(
        _sem_head_kernel,
        grid=(GRID,),
        in_specs=[
            pl.BlockSpec((BLOCK_P, B, C), lambda i: (i, 0, 0)),
            pl.BlockSpec((NUM_CLUSTER, C), lambda i: (0, 0)),
            pl.BlockSpec((1, NUM_CLUSTER), lambda i: (0, 0)),
        ],
        out_specs=pl.BlockSpec((B, NUM_CLUSTER), lambda i: (0, 0)),
        out_shape=jax.ShapeDtypeStruct((B, NUM_CLUSTER), jnp.float32),
        scratch_shapes=[pltpu.VMEM((B, C), jnp.float32)],
    )(planes, W, b2)
